# Initial kernel scaffold; baseline (speedup 1.0000x reference)
import jax
import jax.numpy as jnp
from jax.experimental import pallas as pl


def kernel(x, edge_index_0, edge_index_1, params):
    raise NotImplementedError("write your pallas kernel here")


# speedup vs baseline: 4.1157x; 4.1157x over previous
import functools

import jax
import jax.numpy as jnp
from jax import lax
from jax.experimental import pallas as pl
from jax.experimental.pallas import tpu as pltpu
from jax.experimental.pallas import tpu_sc as plsc

N = 10000
D_IN = 256
H = 64
L = 4
E = 160000
MT = 2

NSUB = 16
CHUNK = 128
NCH = 80
EPT = NCH * CHUNK
EPAD = NSUB * EPT
NPAD = 10240
RPT = NPAD // NSUB

_HIGH = lax.Precision.HIGHEST


def _matT(x, w):
    return lax.dot_general(
        x, w,
        dimension_numbers=(((x.ndim - 1,), (1,)), ((), ())),
        preferred_element_type=jnp.float32,
        precision=_HIGH,
    )


@functools.cache
def _make_agg_sc():
    mesh = plsc.VectorSubcoreMesh(core_axis_name="c", subcore_axis_name="s")
    return pl.kernel(
        _agg_sc_body,
        out_type=jax.ShapeDtypeStruct((MT, NPAD, H), jnp.float32),
        mesh=mesh,
        scratch_types=[
            pltpu.VMEM((NCH, CHUNK), jnp.int32),
            pltpu.VMEM((NCH, CHUNK), jnp.int32),
            pltpu.VMEM((CHUNK, H), jnp.float32),
            pltpu.VMEM((CHUNK, H), jnp.float32),
            pltpu.VMEM_SHARED((NPAD, H), jnp.float32),
            pltpu.SemaphoreType.DMA,
            pltpu.SemaphoreType.DMA,
        ],
        compiler_params=pltpu.CompilerParams(use_tc_tiling_on_sc=False),
    )


def _agg_sc_body(p_hbm, src_hbm, dst_hbm, zeros_hbm, out_hbm,
                 srcv, dstv, rows0, rows1, acc, sem0, sem1):
    c = lax.axis_index("c")
    s = lax.axis_index("s")
    row0 = s * RPT
    pltpu.sync_copy(zeros_hbm.at[pl.ds(row0, RPT)], acc.at[pl.ds(row0, RPT)])
    pltpu.sync_copy(src_hbm.at[c, s], srcv)
    pltpu.sync_copy(dst_hbm.at[c, s], dstv)
    plsc.subcore_barrier()

    def _fire(j, buf, sem):
        pltpu.make_async_copy(p_hbm.at[srcv.at[j]], buf, sem).start()

    def _drain(j, buf, sem):
        pltpu.make_async_copy(p_hbm.at[srcv.at[j]], buf, sem).wait()

    _fire(0, rows0, sem0)

    def body(j2, carry):
        j = 2 * j2
        _fire(j + 1, rows1, sem1)
        _drain(j, rows0, sem0)
        pltpu.sync_copy(rows0, acc.at[dstv.at[j]], add=True)

        @pl.when(j2 != NCH // 2 - 1)
        def _():
            _fire(j + 2, rows0, sem0)

        _drain(j + 1, rows1, sem1)
        pltpu.sync_copy(rows1, acc.at[dstv.at[j + 1]], add=True)
        return carry

    lax.fori_loop(0, NCH // 2, body, 0)
    plsc.subcore_barrier()
    pltpu.sync_copy(acc.at[pl.ds(row0, RPT)], out_hbm.at[c, pl.ds(row0, RPT)])


@functools.cache
def _make_cnt_sc():
    mesh = plsc.VectorSubcoreMesh(core_axis_name="c", subcore_axis_name="s")
    return pl.kernel(
        _cnt_sc_body,
        out_type=jax.ShapeDtypeStruct((MT, NPAD, 16), jnp.float32),
        mesh=mesh,
        scratch_types=[
            pltpu.VMEM((NCH, CHUNK), jnp.int32),
            pltpu.VMEM((CHUNK, 16), jnp.float32),
            pltpu.VMEM_SHARED((NPAD, 16), jnp.float32),
        ],
        compiler_params=pltpu.CompilerParams(use_tc_tiling_on_sc=False),
    )


def _cnt_sc_body(dst_hbm, zeros_hbm, ones_hbm, out_hbm, dstv, onesv, acc):
    c = lax.axis_index("c")
    s = lax.axis_index("s")
    row0 = s * RPT
    pltpu.sync_copy(zeros_hbm.at[pl.ds(row0, RPT)], acc.at[pl.ds(row0, RPT)])
    pltpu.sync_copy(dst_hbm.at[c, s], dstv)
    pltpu.sync_copy(ones_hbm, onesv)
    plsc.subcore_barrier()

    def body(j, carry):
        pltpu.sync_copy(onesv, acc.at[dstv.at[j]], add=True)
        return carry

    lax.fori_loop(0, NCH, body, 0)
    plsc.subcore_barrier()
    pltpu.sync_copy(acc.at[pl.ds(row0, RPT)], out_hbm.at[c, pl.ds(row0, RPT)])


def _fold_body(wd0_ref, ws0_ref, wu0_ref, wdr_ref, wsr_ref, wur_ref,
               bd_ref, bs_ref, bu_ref,
               wa0_ref, wb0_ref, war_ref, wbr_ref, c_ref):
    for m in range(MT):
        u1 = wu0_ref[m, :, :H]
        u2 = wu0_ref[m, :, H:]
        wa0_ref[m] = jnp.dot(u1, wd0_ref[m], precision=_HIGH,
                             preferred_element_type=jnp.float32)
        wb0_ref[m] = jnp.dot(u2, ws0_ref[m], precision=_HIGH,
                             preferred_element_type=jnp.float32)
    for k in range((L - 1) * MT):
        u1 = wur_ref[k, :, :H]
        u2 = wur_ref[k, :, H:]
        war_ref[k] = jnp.dot(u1, wdr_ref[k], precision=_HIGH,
                             preferred_element_type=jnp.float32)
        wbr_ref[k] = jnp.dot(u2, wsr_ref[k], precision=_HIGH,
                             preferred_element_type=jnp.float32)
    for l in range(L):
        for m in range(MT):
            idx = l * MT + m
            if l == 0:
                u1 = wu0_ref[m, :, :H]
                u2 = wu0_ref[m, :, H:]
            else:
                u1 = wur_ref[(l - 1) * MT + m, :, :H]
                u2 = wur_ref[(l - 1) * MT + m, :, H:]
            c_ref[idx:idx + 1] = (_matT(bd_ref[idx:idx + 1], u1)
                                  + _matT(bs_ref[idx:idx + 1], u2)
                                  + bu_ref[idx:idx + 1])


def _fold_call(wd0, ws0, wu0, wdr, wsr, wur, bd, bs, bu):
    return pl.pallas_call(
        _fold_body,
        out_shape=[
            jax.ShapeDtypeStruct((MT, H, D_IN), jnp.float32),
            jax.ShapeDtypeStruct((MT, H, D_IN), jnp.float32),
            jax.ShapeDtypeStruct(((L - 1) * MT, H, H), jnp.float32),
            jax.ShapeDtypeStruct(((L - 1) * MT, H, H), jnp.float32),
            jax.ShapeDtypeStruct((L * MT, H), jnp.float32),
        ],
    )(wd0, ws0, wu0, wdr, wsr, wur, bd, bs, bu)


BN = 1000
NB = N // BN

_full = lambda shape: pl.BlockSpec(shape, lambda i: (0,) * len(shape))
_rows2 = pl.BlockSpec((BN, D_IN), lambda i: (i, 0))
_blk3 = lambda h: pl.BlockSpec((MT, BN, h), lambda i: (0, i, 0))


def _proj0_body(x_ref, wa_ref, wb_ref, pa_ref, pb_ref):
    x = x_ref[:, :]
    for m in range(MT):
        pa_ref[m] = _matT(x, wa_ref[m])
        pb_ref[m] = _matT(x, wb_ref[m])


def _proj0_call(x, wa0, wb0):
    return pl.pallas_call(
        _proj0_body,
        grid=(NB,),
        in_specs=[_rows2, _full((MT, H, D_IN)), _full((MT, H, D_IN))],
        out_specs=[_blk3(H), _blk3(H)],
        out_shape=[
            jax.ShapeDtypeStruct((MT, N, H), jnp.float32),
            jax.ShapeDtypeStruct((MT, N, H), jnp.float32),
        ],
    )(x, wa0, wb0)


def _emb_body(pa_ref, agg_ref, cnt_ref, wa1_ref, ba1_ref, wa2_ref, cv_ref,
              emb_ref, s_ref, ssum_ref, q_ref):
    i = pl.program_id(0)

    @pl.when(i == 0)
    def _():
        s_ref[:, :] = jnp.zeros_like(s_ref)
        ssum_ref[:, :] = jnp.zeros_like(ssum_ref)
        q_ref[:, :] = jnp.zeros_like(q_ref)

    es = []
    for m in range(MT):
        inv = 1.0 / jnp.maximum(cnt_ref[m, :, 0:1], 1.0)
        e = pa_ref[m] + agg_ref[m] * inv + cv_ref[m:m + 1]
        emb_ref[m] = e
        y = jnp.tanh(_matT(e, wa1_ref[:, :]) + ba1_ref[0:1])
        s_ref[m:m + 1] += jnp.sum(y * wa2_ref[0:1], axis=0, keepdims=True)
        ssum_ref[m:m + 1] += jnp.sum(e, axis=0, keepdims=True)
        es.append(e)
    q_ref[0:1] += jnp.sum(es[0] * es[0], axis=0, keepdims=True)
    q_ref[1:2] += jnp.sum(es[0] * es[1], axis=0, keepdims=True)
    q_ref[2:3] += jnp.sum(es[1] * es[1], axis=0, keepdims=True)


def _emb_call(pa, agg, cnt, wa1, ba1, wa2, cv):
    return pl.pallas_call(
        _emb_body,
        grid=(NB,),
        in_specs=[_blk3(H), _blk3(H), _blk3(16), _full((H, H)),
                  _full((1, H)), _full((1, H)), _full((MT, H))],
        out_specs=[_blk3(H), _full((MT, H)), _full((MT, H)), _full((3, H))],
        out_shape=[
            jax.ShapeDtypeStruct((MT, N, H), jnp.float32),
            jax.ShapeDtypeStruct((MT, H), jnp.float32),
            jax.ShapeDtypeStruct((MT, H), jnp.float32),
            jax.ShapeDtypeStruct((3, H), jnp.float32),
        ],
    )(pa, agg, cnt, wa1, ba1, wa2, cv)


def _bn_leaky(emb_ref, s_ref, ssum_ref, q_ref, gb_ref):
    ones = jnp.ones((H, H), jnp.float32)
    s = jnp.dot(s_ref[:, :], ones, precision=_HIGH,
                preferred_element_type=jnp.float32) * (1.0 / N)
    mx = jnp.max(s, axis=0, keepdims=True)
    ex = jnp.exp(s - mx)
    a = ex / jnp.sum(ex, axis=0, keepdims=True)
    a0 = a[0:1]
    a1 = a[1:2]
    mu = (a0 * ssum_ref[0:1] + a1 * ssum_ref[1:2]) * (1.0 / N)
    q = (a0 * a0 * q_ref[0:1] + 2.0 * a0 * a1 * q_ref[1:2]
         + a1 * a1 * q_ref[2:3]) * (1.0 / N)
    var = q - mu * mu
    rstd = lax.rsqrt(var + 1.0)
    hw = a0 * emb_ref[0] + a1 * emb_ref[1]
    hn = (hw - mu) * rstd * gb_ref[0:1] + gb_ref[1:2]
    return jnp.where(hn >= 0, hn, 0.01 * hn)


def _normproj_body(emb_ref, s_ref, ssum_ref, q_ref, gb_ref, wa_ref, wb_ref,
                   pa_ref, pb_ref):
    h = _bn_leaky(emb_ref, s_ref, ssum_ref, q_ref, gb_ref)
    for m in range(MT):
        pa_ref[m] = _matT(h, wa_ref[m])
        pb_ref[m] = _matT(h, wb_ref[m])


def _normproj_call(emb, s, ssum, q, gb, wa, wb):
    return pl.pallas_call(
        _normproj_body,
        grid=(NB,),
        in_specs=[_blk3(H), _full((MT, H)), _full((MT, H)), _full((3, H)),
                  _full((2, H)), _full((MT, H, H)), _full((MT, H, H))],
        out_specs=[_blk3(H), _blk3(H)],
        out_shape=[
            jax.ShapeDtypeStruct((MT, N, H), jnp.float32),
            jax.ShapeDtypeStruct((MT, N, H), jnp.float32),
        ],
    )(emb, s, ssum, q, gb, wa, wb)


def _normfc_body(emb_ref, s_ref, ssum_ref, q_ref, gb_ref, wf_ref, bf_ref,
                 out_ref):
    h = _bn_leaky(emb_ref, s_ref, ssum_ref, q_ref, gb_ref)
    out128 = _matT(h, wf_ref[:, :])
    out_ref[:, :] = out128[:, 0:1] + bf_ref[0, 0]


def _normfc_call(emb, s, ssum, q, gb, wf, bf):
    return pl.pallas_call(
        _normfc_body,
        grid=(NB,),
        in_specs=[_blk3(H), _full((MT, H)), _full((MT, H)), _full((3, H)),
                  _full((2, H)), _full((128, H)),
                  pl.BlockSpec(memory_space=pltpu.SMEM)],
        out_specs=pl.BlockSpec((BN, 1), lambda i: (i, 0)),
        out_shape=jax.ShapeDtypeStruct((N, 1), jnp.float32),
    )(emb, s, ssum, q, gb, wf, bf)


def _agg_call(p_flat, srcg, dstg, zeros64):
    return _make_agg_sc()(p_flat, srcg, dstg, zeros64)


def _cnt_call(dstg, zeros16, ones16):
    return _make_cnt_sc()(dstg, zeros16, ones16)


def _edge_arrays(edge_index_0, edge_index_1):
    srcs, dsts = [], []
    for m, ei in enumerate((edge_index_0, edge_index_1)):
        src = ei[0] + m * N
        dst = ei[1]
        pad = EPAD - E
        src = jnp.concatenate([src, jnp.zeros((pad,), jnp.int32)])
        dst = jnp.concatenate([dst, jnp.full((pad,), N, jnp.int32)])
        srcs.append(src.reshape(NSUB, NCH, CHUNK))
        dsts.append(dst.reshape(NSUB, NCH, CHUNK))
    return jnp.stack(srcs), jnp.stack(dsts)


def kernel(x, edge_index_0, edge_index_1, params):
    layers = params["layers"]

    wd0 = jnp.stack([layers[0]["convs"][m]["lin_dst"]["W"] for m in range(MT)])
    ws0 = jnp.stack([layers[0]["convs"][m]["lin_src"]["W"] for m in range(MT)])
    wu0 = jnp.stack([layers[0]["convs"][m]["lin_update"]["W"] for m in range(MT)])
    wdr = jnp.stack([layers[l]["convs"][m]["lin_dst"]["W"]
                     for l in range(1, L) for m in range(MT)])
    wsr = jnp.stack([layers[l]["convs"][m]["lin_src"]["W"]
                     for l in range(1, L) for m in range(MT)])
    wur = jnp.stack([layers[l]["convs"][m]["lin_update"]["W"]
                     for l in range(1, L) for m in range(MT)])
    bd = jnp.stack([layers[l]["convs"][m]["lin_dst"]["b"]
                    for l in range(L) for m in range(MT)])
    bs = jnp.stack([layers[l]["convs"][m]["lin_src"]["b"]
                    for l in range(L) for m in range(MT)])
    bu = jnp.stack([layers[l]["convs"][m]["lin_update"]["b"]
                    for l in range(L) for m in range(MT)])

    wa0, wb0, war, wbr, cvec = _fold_call(wd0, ws0, wu0, wdr, wsr, wur,
                                          bd, bs, bu)
    war = war.reshape(L - 1, MT, H, H)
    wbr = wbr.reshape(L - 1, MT, H, H)
    cvec = cvec.reshape(L, MT, H)

    srcg, dstg = _edge_arrays(edge_index_0, edge_index_1)
    zeros64 = jnp.zeros((NPAD, H), jnp.float32)
    zeros16 = jnp.zeros((NPAD, 16), jnp.float32)
    ones16 = jnp.ones((CHUNK, 16), jnp.float32)

    cnt = _cnt_call(dstg, zeros16, ones16)
    pa, pb = _proj0_call(x, wa0, wb0)

    for i in range(L):
        agg = _agg_call(pb.reshape(MT * N, H), srcg, dstg, zeros64)
        lp = layers[i]
        emb, s, ssum, q = _emb_call(
            pa, agg, cnt,
            lp["attn1"]["W"], lp["attn1"]["b"].reshape(1, -1),
            lp["attn2"]["W"], cvec[i])
        gb = jnp.stack([lp["bn_gamma"], lp["bn_beta"]])
        if i < L - 1:
            pa, pb = _normproj_call(emb, s, ssum, q, gb, war[i], wbr[i])
        else:
            wf_pad = jnp.pad(params["fc"]["W"], ((0, 127), (0, 0)))
            out = _normfc_call(emb, s, ssum, q, gb, wf_pad,
                               params["fc"]["b"].reshape(1, 1))
    return out
